# baseline (speedup 1.0000x reference)
import functools

import jax
import jax.numpy as jnp
from jax import lax
from jax.experimental import pallas as pl
from jax.experimental.pallas import tpu as pltpu
from jax.experimental.pallas import tpu_sc as plsc

NUM_GRAPHS = 512
NC, NS = 2, 16
GROUPS = 4
STRIPES = NC * NS // GROUPS
CW = 128
CHUNK = 240


def _sc_pool_body(x_hbm, b_hbm, z_hbm, out_hbm,
                  xbuf0, xbuf1, ibuf0, ibuf1, acc,
                  sx0, sx1, si0, si1, *, row_off, n_rows):
    c = lax.axis_index("c")
    s = lax.axis_index("s")
    g = s % GROUPS
    r = c * (NS // GROUPS) + s // GROUPS
    col0 = CW * g

    pltpu.sync_copy(z_hbm.at[:, pl.ds(col0, CW)], acc)

    xbufs, ibufs = (xbuf0, xbuf1), (ibuf0, ibuf1)
    sxs, sis = (sx0, sx1), (si0, si1)

    n_chunks = n_rows // CHUNK
    tail = n_rows - n_chunks * CHUNK
    base_n, extra = n_chunks // STRIPES, n_chunks % STRIPES
    n_r = base_n + (r < extra).astype(jnp.int32)
    n_max = base_n + (1 if extra else 0)

    def start(i, b):
        i_c = jnp.minimum(i, n_r - 1)
        row0 = row_off + (r + STRIPES * i_c) * CHUNK
        pltpu.async_copy(x_hbm.at[pl.ds(row0, CHUNK), pl.ds(col0, CW)],
                         xbufs[b], sxs[b])
        pltpu.async_copy(b_hbm.at[pl.ds(row0, CHUNK)], ibufs[b], sis[b])

    def wait(b):
        pltpu.make_async_copy(x_hbm.at[pl.ds(0, CHUNK), pl.ds(col0, CW)],
                              xbufs[b], sxs[b]).wait()
        pltpu.make_async_copy(b_hbm.at[pl.ds(0, CHUNK)], ibufs[b],
                              sis[b]).wait()

    def consume(xb, ib, count):
        assert count % 16 == 0
        qw = CW // 16

        def tree_sum(vs):
            while len(vs) > 1:
                pairs = [vs[i] + vs[i + 1] for i in range(0, len(vs) - 1, 2)]
                vs = pairs + ([vs[-1]] if len(vs) % 2 else [])
            return vs[0]

        def grp_body(jj, carry):
            bb = 16 * jj
            gidv = ib[pl.ds(bb, 16)]
            g0 = gidv[0]
            uniform = g0 == gidv[15]

            @pl.when(uniform)
            def _():
                for q in range(qw):
                    vs = [xb[bb + t, pl.ds(16 * q, 16)] for t in range(16)]
                    plsc.addupdate(acc.at[g0, pl.ds(16 * q, 16)],
                                   tree_sum(vs))

            @pl.when(jnp.logical_not(uniform))
            def _():
                for lane in range(16):
                    gid = gidv[lane]
                    vals = [xb[bb + lane, pl.ds(16 * q, 16)]
                            for q in range(qw)]
                    for q in range(qw):
                        plsc.addupdate(acc.at[gid, pl.ds(16 * q, 16)],
                                       vals[q])
            return carry

        lax.fori_loop(0, count // 16, grp_body, jnp.int32(0))

    start(jnp.int32(0), 0)
    half = (n_max + 1) // 2

    def pair_body(p, carry):
        i0 = 2 * p
        i1 = i0 + 1
        start(i1, 1)
        wait(0)

        @pl.when(i0 < n_r)
        def _():
            consume(xbufs[0], ibufs[0], CHUNK)

        start(i0 + 2, 0)
        wait(1)

        @pl.when(i1 < n_r)
        def _():
            consume(xbufs[1], ibufs[1], CHUNK)
        return carry

    lax.fori_loop(0, half, pair_body, jnp.int32(0))
    wait(0)

    if tail:
        @pl.when(r == STRIPES - 1)
        def _():
            t0 = row_off + n_chunks * CHUNK
            pltpu.sync_copy(x_hbm.at[pl.ds(t0, tail), pl.ds(col0, CW)],
                            xbuf0.at[pl.ds(0, tail)])
            pltpu.sync_copy(b_hbm.at[pl.ds(t0, tail)],
                            ibuf0.at[pl.ds(0, tail)])
            consume(xbuf0, ibuf0, tail)

    pltpu.sync_copy(acc, out_hbm.at[r, :, pl.ds(col0, CW)])


def _sc_pool(x, batch32, zeros, row_off):
    n, d = x.shape
    body = functools.partial(_sc_pool_body, row_off=row_off,
                             n_rows=n - row_off)
    return pl.kernel(
        body,
        out_type=jax.ShapeDtypeStruct((STRIPES, NUM_GRAPHS, d), jnp.float32),
        mesh=plsc.VectorSubcoreMesh(core_axis_name="c", subcore_axis_name="s"),
        scratch_types=[
            pltpu.VMEM((CHUNK, CW), jnp.float32),
            pltpu.VMEM((CHUNK, CW), jnp.float32),
            pltpu.VMEM((CHUNK,), jnp.int32),
            pltpu.VMEM((CHUNK,), jnp.int32),
            pltpu.VMEM((NUM_GRAPHS, CW), jnp.float32),
            pltpu.SemaphoreType.DMA,
            pltpu.SemaphoreType.DMA,
            pltpu.SemaphoreType.DMA,
            pltpu.SemaphoreType.DMA,
        ],
    )(x, batch32, zeros)


BN_TC = 2000


def _tc_pool_kernel(batch_ref, x_ref, out_ref, *, bn):
    i = pl.program_id(0)
    ids = batch_ref[0, 0, :]
    onehot = (ids[:, None]
              == lax.broadcasted_iota(jnp.int32, (bn, NUM_GRAPHS), 1)
              ).astype(jnp.bfloat16)
    xf = x_ref[...]
    x_hi = xf.astype(jnp.bfloat16)
    x_lo = (xf - x_hi.astype(jnp.float32)).astype(jnp.bfloat16)
    dims = (((0,), (0,)), ((), ()))
    partial = (lax.dot_general(onehot, x_hi, dims,
                               preferred_element_type=jnp.float32)
               + lax.dot_general(onehot, x_lo, dims,
                                 preferred_element_type=jnp.float32))

    @pl.when(i == 0)
    def _():
        out_ref[...] = partial

    @pl.when(i > 0)
    def _():
        out_ref[...] += partial


def _tc_pool(x, batch32, n_tc):
    n, d = x.shape
    nb = n_tc // BN_TC
    b2 = batch32[:n_tc].reshape(nb, 1, BN_TC)
    return pl.pallas_call(
        functools.partial(_tc_pool_kernel, bn=BN_TC),
        grid=(nb,),
        in_specs=[pl.BlockSpec((1, 1, BN_TC), lambda i: (i, 0, 0)),
                  pl.BlockSpec((BN_TC, d), lambda i: (i, 0))],
        out_specs=pl.BlockSpec((NUM_GRAPHS, d), lambda i: (0, 0)),
        out_shape=jax.ShapeDtypeStruct((NUM_GRAPHS, d), jnp.float32),
    )(b2, x)


def _mlp_kernel(ptc_ref, p_ref, w0_ref, b0_ref, w1_ref, b1_ref, w2_ref, b2_ref,
                w3_ref, b3_ref, out_ref):
    g = jnp.sum(p_ref[...], axis=0) + ptc_ref[...]
    h = jnp.maximum(
        jnp.dot(g, w0_ref[...], preferred_element_type=jnp.float32)
        + b0_ref[...], 0.0)
    h = jnp.maximum(
        jnp.dot(h, w1_ref[...], preferred_element_type=jnp.float32)
        + b1_ref[...], 0.0)
    h = jnp.maximum(
        jnp.dot(h, w2_ref[...], preferred_element_type=jnp.float32)
        + b2_ref[...], 0.0)
    out_ref[...] = (jnp.dot(h, w3_ref[...],
                            preferred_element_type=jnp.float32)
                    + b3_ref[...])


def kernel(x, batch, W0, b0, W1, b1, W2, b2, W3, b3):
    n, d = x.shape
    batch32 = batch.astype(jnp.int32)
    zeros = jnp.zeros((NUM_GRAPHS, d), jnp.float32)

    n_tc = (11 * n // 25) // BN_TC * BN_TC
    partials = _sc_pool(x, batch32, zeros, n_tc)
    ptc = _tc_pool(x, batch32, n_tc)

    w3p = jnp.zeros((W3.shape[0], 128), jnp.float32).at[:, :1].set(W3)
    b3p = jnp.zeros((1, 128), jnp.float32).at[:, :1].set(b3[None, :])

    full = lambda sh: pl.BlockSpec(sh, lambda: (0,) * len(sh))
    out = pl.pallas_call(
        _mlp_kernel,
        in_specs=[
            full(ptc.shape),
            full(partials.shape),
            full(W0.shape), full((1, b0.shape[0])),
            full(W1.shape), full((1, b1.shape[0])),
            full(W2.shape), full((1, b2.shape[0])),
            full(w3p.shape), full(b3p.shape),
        ],
        out_specs=pl.BlockSpec((NUM_GRAPHS, 128), lambda: (0, 0)),
        out_shape=jax.ShapeDtypeStruct((NUM_GRAPHS, 128), jnp.float32),
    )(ptc, partials, W0, b0[None, :], W1, b1[None, :], W2, b2[None, :], w3p, b3p)
    return out[:, :1]


# speedup vs baseline: 1.0966x; 1.0966x over previous
import functools

import jax
import jax.numpy as jnp
from jax import lax
from jax.experimental import pallas as pl
from jax.experimental.pallas import tpu as pltpu
from jax.experimental.pallas import tpu_sc as plsc

NUM_GRAPHS = 512
NC, NS = 2, 16
GROUPS = 4
STRIPES = NC * NS // GROUPS
CW = 128
CHUNK = 240


def _sc_pool_body(x_hbm, b_hbm, z_hbm, out_hbm,
                  xbuf0, xbuf1, ibuf0, ibuf1, acc,
                  sx0, sx1, si0, si1, *, row_off, n_rows):
    c = lax.axis_index("c")
    s = lax.axis_index("s")
    g = s % GROUPS
    r = c * (NS // GROUPS) + s // GROUPS
    col0 = CW * g

    pltpu.sync_copy(z_hbm.at[:, pl.ds(col0, CW)], acc)

    xbufs, ibufs = (xbuf0, xbuf1), (ibuf0, ibuf1)
    sxs, sis = (sx0, sx1), (si0, si1)

    n_chunks = n_rows // CHUNK
    tail = n_rows - n_chunks * CHUNK
    base_n, extra = n_chunks // STRIPES, n_chunks % STRIPES
    n_r = base_n + (r < extra).astype(jnp.int32)
    n_max = base_n + (1 if extra else 0)

    def start(i, b):
        i_c = jnp.minimum(i, n_r - 1)
        row0 = row_off + (r + STRIPES * i_c) * CHUNK
        pltpu.async_copy(x_hbm.at[pl.ds(row0, CHUNK), pl.ds(col0, CW)],
                         xbufs[b], sxs[b])
        pltpu.async_copy(b_hbm.at[pl.ds(row0, CHUNK)], ibufs[b], sis[b])

    def wait(b):
        pltpu.make_async_copy(x_hbm.at[pl.ds(0, CHUNK), pl.ds(col0, CW)],
                              xbufs[b], sxs[b]).wait()
        pltpu.make_async_copy(b_hbm.at[pl.ds(0, CHUNK)], ibufs[b],
                              sis[b]).wait()

    def consume(xb, ib, count):
        assert count % 16 == 0
        qw = CW // 16

        def tree_sum(vs):
            while len(vs) > 1:
                pairs = [vs[i] + vs[i + 1] for i in range(0, len(vs) - 1, 2)]
                vs = pairs + ([vs[-1]] if len(vs) % 2 else [])
            return vs[0]

        def grp_body(jj, carry):
            bb = 16 * jj
            gidv = ib[pl.ds(bb, 16)]
            g0 = gidv[0]
            uniform = g0 == gidv[15]

            @pl.when(uniform)
            def _():
                for q in range(qw):
                    vs = [xb[bb + t, pl.ds(16 * q, 16)] for t in range(16)]
                    plsc.addupdate(acc.at[g0, pl.ds(16 * q, 16)],
                                   tree_sum(vs))

            @pl.when(jnp.logical_not(uniform))
            def _():
                for lane in range(16):
                    gid = gidv[lane]
                    vals = [xb[bb + lane, pl.ds(16 * q, 16)]
                            for q in range(qw)]
                    for q in range(qw):
                        plsc.addupdate(acc.at[gid, pl.ds(16 * q, 16)],
                                       vals[q])
            return carry

        lax.fori_loop(0, count // 16, grp_body, jnp.int32(0))

    start(jnp.int32(0), 0)
    half = (n_max + 1) // 2

    def pair_body(p, carry):
        i0 = 2 * p
        i1 = i0 + 1
        start(i1, 1)
        wait(0)

        @pl.when(i0 < n_r)
        def _():
            consume(xbufs[0], ibufs[0], CHUNK)

        start(i0 + 2, 0)
        wait(1)

        @pl.when(i1 < n_r)
        def _():
            consume(xbufs[1], ibufs[1], CHUNK)
        return carry

    lax.fori_loop(0, half, pair_body, jnp.int32(0))
    wait(0)

    if tail:
        @pl.when(r == STRIPES - 1)
        def _():
            t0 = row_off + n_chunks * CHUNK
            pltpu.sync_copy(x_hbm.at[pl.ds(t0, tail), pl.ds(col0, CW)],
                            xbuf0.at[pl.ds(0, tail)])
            pltpu.sync_copy(b_hbm.at[pl.ds(t0, tail)],
                            ibuf0.at[pl.ds(0, tail)])
            consume(xbuf0, ibuf0, tail)

    pltpu.sync_copy(acc, out_hbm.at[r, :, pl.ds(col0, CW)])


def _sc_pool(x, batch32, zeros, row_off):
    n, d = x.shape
    body = functools.partial(_sc_pool_body, row_off=row_off,
                             n_rows=n - row_off)
    return pl.kernel(
        body,
        out_type=jax.ShapeDtypeStruct((STRIPES, NUM_GRAPHS, d), jnp.float32),
        mesh=plsc.VectorSubcoreMesh(core_axis_name="c", subcore_axis_name="s"),
        scratch_types=[
            pltpu.VMEM((CHUNK, CW), jnp.float32),
            pltpu.VMEM((CHUNK, CW), jnp.float32),
            pltpu.VMEM((CHUNK,), jnp.int32),
            pltpu.VMEM((CHUNK,), jnp.int32),
            pltpu.VMEM((NUM_GRAPHS, CW), jnp.float32),
            pltpu.SemaphoreType.DMA,
            pltpu.SemaphoreType.DMA,
            pltpu.SemaphoreType.DMA,
            pltpu.SemaphoreType.DMA,
        ],
    )(x, batch32, zeros)


BN_TC = 2000


def _tc_pool_kernel(batch_ref, x_ref, out_ref, *, bn):
    i = pl.program_id(0)
    ids = batch_ref[0, 0, :]
    onehot = (ids[:, None]
              == lax.broadcasted_iota(jnp.int32, (bn, NUM_GRAPHS), 1)
              ).astype(jnp.bfloat16)
    xf = x_ref[...]
    x_hi = xf.astype(jnp.bfloat16)
    x_lo = (xf - x_hi.astype(jnp.float32)).astype(jnp.bfloat16)
    dims = (((0,), (0,)), ((), ()))
    partial = (lax.dot_general(onehot, x_hi, dims,
                               preferred_element_type=jnp.float32)
               + lax.dot_general(onehot, x_lo, dims,
                                 preferred_element_type=jnp.float32))

    @pl.when(i == 0)
    def _():
        out_ref[...] = partial

    @pl.when(i > 0)
    def _():
        out_ref[...] += partial


def _tc_pool(x, batch32, n_tc):
    n, d = x.shape
    nb = n_tc // BN_TC
    b2 = batch32[:n_tc].reshape(nb, 1, BN_TC)
    return pl.pallas_call(
        functools.partial(_tc_pool_kernel, bn=BN_TC),
        grid=(nb,),
        in_specs=[pl.BlockSpec((1, 1, BN_TC), lambda i: (i, 0, 0)),
                  pl.BlockSpec((BN_TC, d), lambda i: (i, 0))],
        out_specs=pl.BlockSpec((NUM_GRAPHS, d), lambda i: (0, 0)),
        out_shape=jax.ShapeDtypeStruct((NUM_GRAPHS, d), jnp.float32),
    )(b2, x)


def _mlp_kernel(ptc_ref, p_ref, w0_ref, b0_ref, w1_ref, b1_ref, w2_ref, b2_ref,
                w3_ref, b3_ref, out_ref):
    g = jnp.sum(p_ref[...], axis=0) + ptc_ref[...]
    h = jnp.maximum(
        jnp.dot(g, w0_ref[...], preferred_element_type=jnp.float32)
        + b0_ref[...], 0.0)
    h = jnp.maximum(
        jnp.dot(h, w1_ref[...], preferred_element_type=jnp.float32)
        + b1_ref[...], 0.0)
    h = jnp.maximum(
        jnp.dot(h, w2_ref[...], preferred_element_type=jnp.float32)
        + b2_ref[...], 0.0)
    out_ref[...] = (jnp.dot(h, w3_ref[...],
                            preferred_element_type=jnp.float32)
                    + b3_ref[...])


def kernel(x, batch, W0, b0, W1, b1, W2, b2, W3, b3):
    n, d = x.shape
    batch32 = batch.astype(jnp.int32)
    zeros = jnp.zeros((NUM_GRAPHS, d), jnp.float32)

    n_tc = (14 * n // 25) // BN_TC * BN_TC
    partials = _sc_pool(x, batch32, zeros, n_tc)
    ptc = _tc_pool(x, batch32, n_tc)

    w3p = jnp.zeros((W3.shape[0], 128), jnp.float32).at[:, :1].set(W3)
    b3p = jnp.zeros((1, 128), jnp.float32).at[:, :1].set(b3[None, :])

    full = lambda sh: pl.BlockSpec(sh, lambda: (0,) * len(sh))
    out = pl.pallas_call(
        _mlp_kernel,
        in_specs=[
            full(ptc.shape),
            full(partials.shape),
            full(W0.shape), full((1, b0.shape[0])),
            full(W1.shape), full((1, b1.shape[0])),
            full(W2.shape), full((1, b2.shape[0])),
            full(w3p.shape), full(b3p.shape),
        ],
        out_specs=pl.BlockSpec((NUM_GRAPHS, 128), lambda: (0, 0)),
        out_shape=jax.ShapeDtypeStruct((NUM_GRAPHS, 128), jnp.float32),
    )(ptc, partials, W0, b0[None, :], W1, b1[None, :], W2, b2[None, :], w3p, b3p)
    return out[:, :1]
